# baseline (speedup 1.0000x reference)
import functools

import jax
import jax.numpy as jnp
from jax import lax
from jax.experimental import pallas as pl
from jax.experimental.pallas import tpu as pltpu
from jax.experimental.pallas import tpu_sc as plsc

S = 16384
NC = 2
NS = 16
L = 16
NW = NC * NS
ROWS = 128
COLS = S // ROWS


def _sc_kernel(n8, per_w, t):
    groups = t // L

    @functools.partial(
        pl.kernel,
        out_type=jax.ShapeDtypeStruct((NC, ROWS, COLS), jnp.float32),
        mesh=plsc.VectorSubcoreMesh(
            core_axis_name="c", subcore_axis_name="s",
            num_cores=NC, num_subcores=NS),
        compiler_params=pltpu.CompilerParams(needs_layout_passes=False),
        scratch_types=[
            pltpu.VMEM((2 * t,), jnp.float32),
            pltpu.VMEM((t,), jnp.float32),
            pltpu.VMEM((S,), jnp.float32),
            pltpu.VMEM((ROWS, COLS), jnp.float32),
            pltpu.VMEM((ROWS,), jnp.int32),
            pltpu.VMEM_SHARED((ROWS, COLS), jnp.float32),
            pltpu.SemaphoreType.DMA,
        ],
    )
    def k(ab_hbm, val_hbm, x_hbm, out_hbm, ab_v, val_v, x_v, y_v, rows_v,
          y_shared, sem):
        c = lax.axis_index("c")
        s = lax.axis_index("s")
        wid = c * NS + s
        base = jnp.minimum(wid * per_w, n8 - t)
        lo = wid * per_w - base
        hi = jnp.minimum((wid + 1) * per_w, n8) - base

        d_x = pltpu.async_copy(x_hbm, x_v, sem)
        d_ab = pltpu.async_copy(ab_hbm.at[pl.ds(2 * base, 2 * t)], ab_v, sem)
        d_v = pltpu.async_copy(val_hbm.at[pl.ds(base, t)], val_v, sem)

        zeros16 = jnp.zeros((L,), jnp.float32)
        iota16 = lax.iota(jnp.int32, L)

        def zero_body(i, _):
            y_v[i >> 3, pl.ds((i & 7) * L, L)] = zeros16
            return 0

        lax.fori_loop(0, ROWS * (COLS // L), zero_body, 0)

        def iota_body(i, _):
            rows_v[pl.ds(i * L, L)] = iota16 + i * L
            return 0

        lax.fori_loop(0, ROWS // L, iota_body, 0)

        @pl.when(s == 0)
        def _():
            pltpu.sync_copy(y_v, y_shared)

        d_x.wait()
        d_ab.wait()
        d_v.wait()
        plsc.subcore_barrier()

        one16 = jnp.ones((L,), jnp.int32)
        zero16 = jnp.zeros((L,), jnp.int32)
        fone16 = jnp.ones((L,), jnp.float32)

        @plsc.parallel_loop(0, groups, 1, unroll=4)
        def _(g):
            ivec = iota16 + g * L
            m = (ivec >= lo) & (ivec < hi)
            av = plsc.load_gather(ab_v, [ivec * 2], mask=m)
            bv = plsc.load_gather(ab_v, [ivec * 2 + 1], mask=m)
            v = val_v[pl.ds(g * L, L)]
            fai = av.astype(jnp.int32)
            fa = fai.astype(jnp.float32)
            ta = av - fa
            ma = av > fa
            cai = fai + jnp.where(ma, one16, zero16)
            fbi = bv.astype(jnp.int32)
            fb = fbi.astype(jnp.float32)
            tb = bv - fb
            mb = bv > fb
            cbi = fbi + jnp.where(mb, one16, zero16)

            xf = plsc.load_gather(x_v, [fbi], mask=m)
            xc = plsc.load_gather(x_v, [cbi], mask=m)

            t0 = v * ((1.0 - tb) * xf + jnp.where(mb, tb, fone16) * xc)
            sf = (1.0 - ta) * t0
            sc = jnp.where(ma, ta, fone16) * t0

            plsc.addupdate_scatter(
                y_v, [fai >> 7, fai & (COLS - 1)], sf, mask=m)
            plsc.addupdate_scatter(
                y_v, [cai >> 7, cai & (COLS - 1)], sc, mask=m)

        pltpu.sync_copy(y_v, y_shared.at[rows_v], add=True)
        plsc.subcore_barrier()

        @pl.when(s == 0)
        def _():
            pltpu.sync_copy(y_shared, out_hbm.at[c])

    return k


def _combine(n_tail):
    def body(p_ref, x_ref, tab_ref, tval_ref, o_ref):
        y = p_ref[0] + p_ref[1]
        rows = lax.broadcasted_iota(jnp.int32, (ROWS, COLS), 0)
        cols = lax.broadcasted_iota(jnp.int32, (ROWS, COLS), 1)
        for e in range(n_tail):
            a = tab_ref[e, 0]
            b = tab_ref[e, 1]
            v = tval_ref[e]
            fai = a.astype(jnp.int32)
            fa = fai.astype(jnp.float32)
            ta = a - fa
            ma = a > fa
            cai = fai + jnp.where(ma, 1, 0)
            fbi = b.astype(jnp.int32)
            fb = fbi.astype(jnp.float32)
            tb = b - fb
            mb = b > fb
            cbi = fbi + jnp.where(mb, 1, 0)
            xf = jnp.sum(jnp.where(
                (rows == (fbi >> 7)) & (cols == (fbi & (COLS - 1))),
                x_ref[...], 0.0))
            xc = jnp.sum(jnp.where(
                (rows == (cbi >> 7)) & (cols == (cbi & (COLS - 1))),
                x_ref[...], 0.0))
            t0 = v * ((1.0 - tb) * xf + jnp.where(mb, tb, 1.0) * xc)
            y = y + jnp.where(
                (rows == (fai >> 7)) & (cols == (fai & (COLS - 1))),
                (1.0 - ta) * t0, 0.0)
            y = y + jnp.where(
                (rows == (cai >> 7)) & (cols == (cai & (COLS - 1))),
                jnp.where(ma, ta, 1.0) * t0, 0.0)
        o_ref[...] = y

    return body


def kernel(input, real_indices, real_values):
    n = real_indices.shape[0]
    n8 = (n // 8) * 8
    n_tail = n - n8
    per_w = ((n8 + NW - 1) // NW + 7) // 8 * 8
    t = (per_w + L - 1) // L * L

    ab_flat = real_indices.reshape(-1)
    partials = _sc_kernel(n8, per_w, t)(ab_flat, real_values, input)

    tab = real_indices[n8:]
    tval = real_values[n8:]
    y = pl.pallas_call(
        _combine(n_tail),
        in_specs=[
            pl.BlockSpec(memory_space=pltpu.VMEM),
            pl.BlockSpec(memory_space=pltpu.VMEM),
            pl.BlockSpec(memory_space=pltpu.SMEM),
            pl.BlockSpec(memory_space=pltpu.SMEM),
        ],
        out_shape=jax.ShapeDtypeStruct((ROWS, COLS), jnp.float32),
    )(partials, input.reshape(ROWS, COLS), tab, tval)
    return y.reshape(S)
